# Initial kernel scaffold; baseline (speedup 1.0000x reference)
import jax
import jax.numpy as jnp
from jax.experimental import pallas as pl


def kernel(timesteps, W, base, w_ini, base_ini, min_diff):
    raise NotImplementedError("write your pallas kernel here")


# speedup vs baseline: 4.1140x; 4.1140x over previous
import functools

import jax
import jax.numpy as jnp
from jax import lax
from jax.experimental import pallas as pl
from jax.experimental.pallas import tpu as pltpu
from jax.experimental.pallas import tpu_sc as plsc

_L = 16
_T_PAD = 1024
_N_CHUNKS = _T_PAD // _L
_B = 16384
_NW = 32
_B_PER_W = _B // _NW
_G_CHUNKS = _B_PER_W // _L

_mesh = plsc.VectorSubcoreMesh(core_axis_name="c", subcore_axis_name="s")


@functools.partial(
    pl.kernel,
    mesh=_mesh,
    out_type=jax.ShapeDtypeStruct((_B,), jnp.float32),
    compiler_params=pltpu.CompilerParams(needs_layout_passes=False),
    scratch_types=[
        pltpu.VMEM((_T_PAD,), jnp.float32),
        pltpu.VMEM((3 * _L,), jnp.float32),
        pltpu.VMEM((_B_PER_W,), jnp.int32),
        pltpu.VMEM((_B_PER_W,), jnp.float32),
    ],
)
def _snr_lookup(ts_hbm, w_hbm, scal_hbm, out_hbm, tab_v, scal_v, idx_v, out_v):
    wid = lax.axis_index("s") * 2 + lax.axis_index("c")
    base = wid * _B_PER_W
    pltpu.sync_copy(w_hbm, tab_v)
    pltpu.sync_copy(scal_hbm, scal_v)
    pltpu.sync_copy(ts_hbm.at[pl.ds(base, _B_PER_W)], idx_v)

    wini = scal_v[pl.ds(0, _L)]
    md = scal_v[pl.ds(_L, _L)]
    carry = scal_v[pl.ds(2 * _L, _L)]

    for i in range(_N_CHUNKS):
        x = tab_v[pl.ds(i * _L, _L)] + wini
        s = 1.0 / (1.0 + jnp.exp(-x)) + md
        tab_v[pl.ds(i * _L, _L)] = plsc.cumsum(s) + carry
        carry = carry + lax.broadcast(jnp.sum(s), (_L,))

    for j in range(_G_CHUNKS):
        ids = idx_v[pl.ds(j * _L, _L)]
        out_v[pl.ds(j * _L, _L)] = plsc.load_gather(tab_v, [ids])

    pltpu.sync_copy(out_v, out_hbm.at[pl.ds(base, _B_PER_W)])


def kernel(timesteps, W, base, w_ini, base_ini, min_diff):
    w_pad = jnp.zeros((_T_PAD,), jnp.float32).at[: W.shape[0]].set(W)
    scal = jnp.concatenate([
        jnp.full((_L,), w_ini, jnp.float32),
        jnp.full((_L,), min_diff, jnp.float32),
        jnp.full((_L,), -base_ini, jnp.float32) - base.astype(jnp.float32)[0],
    ])
    return _snr_lookup(timesteps.astype(jnp.int32), w_pad, scal)


# speedup vs baseline: 4.2476x; 1.0325x over previous
import functools

import jax
import jax.numpy as jnp
from jax import lax
from jax.experimental import pallas as pl
from jax.experimental.pallas import tpu as pltpu
from jax.experimental.pallas import tpu_sc as plsc

_L = 16
_T_PAD = 1024
_N_CHUNKS = _T_PAD // _L
_B = 16384
_NW = 32
_B_PER_W = _B // _NW
_G_CHUNKS = _B_PER_W // _L

_BUILD_UNROLL = 8
_GATHER_UNROLL = 8

_mesh = plsc.VectorSubcoreMesh(core_axis_name="c", subcore_axis_name="s")


@functools.partial(
    pl.kernel,
    mesh=_mesh,
    out_type=jax.ShapeDtypeStruct((_B,), jnp.float32),
    compiler_params=pltpu.CompilerParams(needs_layout_passes=False),
    scratch_types=[
        pltpu.VMEM((_T_PAD,), jnp.float32),
        pltpu.VMEM((3 * _L,), jnp.float32),
        pltpu.VMEM((_B_PER_W,), jnp.int32),
        pltpu.VMEM((_B_PER_W,), jnp.float32),
    ],
)
def _snr_lookup(ts_hbm, w_hbm, scal_hbm, out_hbm, tab_v, scal_v, idx_v, out_v):
    wid = lax.axis_index("s") * 2 + lax.axis_index("c")
    base = wid * _B_PER_W
    pltpu.sync_copy(w_hbm, tab_v)
    pltpu.sync_copy(scal_hbm, scal_v)
    pltpu.sync_copy(ts_hbm.at[pl.ds(base, _B_PER_W)], idx_v)

    wini = scal_v[pl.ds(0, _L)]
    md = scal_v[pl.ds(_L, _L)]
    carry0 = scal_v[pl.ds(2 * _L, _L)]

    def build_chunk(i, carry):
        x = tab_v[pl.ds(i * _L, _L)] + wini
        s = 1.0 / (1.0 + jnp.exp(-x)) + md
        tab_v[pl.ds(i * _L, _L)] = plsc.cumsum(s) + carry
        return carry + lax.broadcast(jnp.sum(s), (_L,))

    def build_body(i, carry):
        for u in range(_BUILD_UNROLL):
            carry = build_chunk(i * _BUILD_UNROLL + u, carry)
        return carry

    lax.fori_loop(0, _N_CHUNKS // _BUILD_UNROLL, build_body, carry0)

    def gather_body(j, _):
        for u in range(_GATHER_UNROLL):
            k = j * _GATHER_UNROLL + u
            ids = idx_v[pl.ds(k * _L, _L)]
            out_v[pl.ds(k * _L, _L)] = plsc.load_gather(tab_v, [ids])
        return 0

    lax.fori_loop(0, _G_CHUNKS // _GATHER_UNROLL, gather_body, 0)

    pltpu.sync_copy(out_v, out_hbm.at[pl.ds(base, _B_PER_W)])


def kernel(timesteps, W, base, w_ini, base_ini, min_diff):
    w_pad = jnp.zeros((_T_PAD,), jnp.float32).at[: W.shape[0]].set(W)
    scal = jnp.concatenate([
        jnp.full((_L,), w_ini, jnp.float32),
        jnp.full((_L,), min_diff, jnp.float32),
        jnp.full((_L,), -base_ini, jnp.float32) - base.astype(jnp.float32)[0],
    ])
    return _snr_lookup(timesteps.astype(jnp.int32), w_pad, scal)


# speedup vs baseline: 4.5944x; 1.0816x over previous
import functools

import jax
import jax.numpy as jnp
from jax import lax
from jax.experimental import pallas as pl
from jax.experimental.pallas import tpu as pltpu
from jax.experimental.pallas import tpu_sc as plsc

_L = 16
_T = 1001
_T_PAD = 1024
_N_CHUNKS = _T_PAD // _L
_B = 16384
_NW = 32
_B_PER_W = _B // _NW
_G_CHUNKS = _B_PER_W // _L
_BUILD_UNROLL = 8
_GATHER_UNROLL = 8

_mesh = plsc.VectorSubcoreMesh(core_axis_name="c", subcore_axis_name="s")


@functools.partial(
    pl.kernel,
    mesh=_mesh,
    out_type=jax.ShapeDtypeStruct((_B,), jnp.float32),
    compiler_params=pltpu.CompilerParams(needs_layout_passes=False),
    scratch_types=[
        pltpu.VMEM((_T_PAD,), jnp.float32),
        pltpu.VMEM((_L,), jnp.float32),
        pltpu.VMEM((_B_PER_W,), jnp.int32),
        pltpu.VMEM((_B_PER_W,), jnp.float32),
        pltpu.SemaphoreType.DMA,
        pltpu.SemaphoreType.DMA,
        pltpu.SemaphoreType.DMA,
    ],
)
def _snr_lookup(ts_hbm, w_hbm, scal_hbm, out_hbm,
                tab_v, scal_v, idx_v, out_v, sem0, sem1, sem2):
    wid = lax.axis_index("s") * 2 + lax.axis_index("c")
    off = wid * _B_PER_W
    cw = pltpu.async_copy(w_hbm, tab_v.at[pl.ds(0, _T)], sem0)
    cs = pltpu.async_copy(scal_hbm, scal_v.at[pl.ds(0, 4)], sem1)
    ci = pltpu.async_copy(ts_hbm.at[pl.ds(off, _B_PER_W)], idx_v, sem2)

    cs.wait()
    sv = scal_v[...]
    wini = lax.broadcast(sv[0], (_L,))
    md = lax.broadcast(sv[1], (_L,))
    carry0 = lax.broadcast(-(sv[2] + sv[3]), (_L,))
    cw.wait()

    def build_chunk(i, carry):
        x = tab_v[pl.ds(i * _L, _L)] + wini
        s = 1.0 / (1.0 + jnp.exp(-x)) + md
        tab_v[pl.ds(i * _L, _L)] = plsc.cumsum(s) + carry
        return carry + lax.broadcast(jnp.sum(s), (_L,))

    def build_body(i, carry):
        for u in range(_BUILD_UNROLL):
            carry = build_chunk(i * _BUILD_UNROLL + u, carry)
        return carry

    lax.fori_loop(0, _N_CHUNKS // _BUILD_UNROLL, build_body, carry0)
    ci.wait()

    def gather_body(j, _):
        for u in range(_GATHER_UNROLL):
            k = j * _GATHER_UNROLL + u
            ids = idx_v[pl.ds(k * _L, _L)]
            out_v[pl.ds(k * _L, _L)] = plsc.load_gather(tab_v, [ids])
        return 0

    lax.fori_loop(0, _G_CHUNKS // _GATHER_UNROLL, gather_body, 0)

    pltpu.sync_copy(out_v, out_hbm.at[pl.ds(off, _B_PER_W)])


def kernel(timesteps, W, base, w_ini, base_ini, min_diff):
    scal = jnp.stack([w_ini, min_diff, base_ini, base[0]]).astype(jnp.float32)
    return _snr_lookup(timesteps, W, scal)


# speedup vs baseline: 4.6949x; 1.0219x over previous
import functools

import jax
import jax.numpy as jnp
from jax import lax
from jax.experimental import pallas as pl
from jax.experimental.pallas import tpu as pltpu
from jax.experimental.pallas import tpu_sc as plsc

_L = 16
_T = 1001
_T_PAD = 1024
_N_CHUNKS = _T_PAD // _L
_B = 16384
_NW = 32
_B_PER_W = _B // _NW
_G_CHUNKS = _B_PER_W // _L
_BUILD_UNROLL = 8
_GATHER_UNROLL = 8

_mesh = plsc.VectorSubcoreMesh(core_axis_name="c", subcore_axis_name="s")


@functools.partial(
    pl.kernel,
    mesh=_mesh,
    out_type=jax.ShapeDtypeStruct((_B,), jnp.float32),
    compiler_params=pltpu.CompilerParams(needs_layout_passes=False),
    scratch_types=[
        pltpu.VMEM((_T_PAD,), jnp.float32),
        pltpu.VMEM((2 * _L,), jnp.float32),
        pltpu.VMEM((_B_PER_W,), jnp.int32),
        pltpu.VMEM((_B_PER_W,), jnp.float32),
        pltpu.SemaphoreType.DMA,
        pltpu.SemaphoreType.DMA,
        pltpu.SemaphoreType.DMA,
    ],
)
def _snr_lookup(ts_hbm, w_hbm, wini_hbm, md_hbm, bini_hbm, base_hbm, out_hbm,
                tab_v, scal_v, idx_v, out_v, sem0, sem1, sem2):
    wid = lax.axis_index("s") * 2 + lax.axis_index("c")
    off = wid * _B_PER_W
    cw = pltpu.async_copy(w_hbm, tab_v.at[pl.ds(0, _T)], sem0)
    c0 = pltpu.async_copy(wini_hbm, scal_v.at[pl.ds(0, 1)], sem1)
    c1 = pltpu.async_copy(md_hbm, scal_v.at[pl.ds(8, 1)], sem1)
    c2 = pltpu.async_copy(bini_hbm, scal_v.at[pl.ds(16, 1)], sem1)
    c3 = pltpu.async_copy(base_hbm, scal_v.at[pl.ds(24, 1)], sem1)
    ci = pltpu.async_copy(ts_hbm.at[pl.ds(off, _B_PER_W)], idx_v, sem2)

    c0.wait()
    c1.wait()
    c2.wait()
    c3.wait()
    sv0 = scal_v[pl.ds(0, _L)]
    sv1 = scal_v[pl.ds(_L, _L)]
    wini = lax.broadcast(sv0[0], (_L,))
    md = lax.broadcast(sv0[8], (_L,))
    carry0 = lax.broadcast(-(sv1[0] + sv1[8]), (_L,))
    cw.wait()

    def build_chunk(i, carry):
        x = tab_v[pl.ds(i * _L, _L)] + wini
        s = 1.0 / (1.0 + jnp.exp(-x)) + md
        tab_v[pl.ds(i * _L, _L)] = plsc.cumsum(s) + carry
        return carry + lax.broadcast(jnp.sum(s), (_L,))

    def build_body(i, carry):
        for u in range(_BUILD_UNROLL):
            carry = build_chunk(i * _BUILD_UNROLL + u, carry)
        return carry

    lax.fori_loop(0, _N_CHUNKS // _BUILD_UNROLL, build_body, carry0)
    ci.wait()

    def gather_body(j, _):
        for u in range(_GATHER_UNROLL):
            k = j * _GATHER_UNROLL + u
            ids = idx_v[pl.ds(k * _L, _L)]
            out_v[pl.ds(k * _L, _L)] = plsc.load_gather(tab_v, [ids])
        return 0

    lax.fori_loop(0, _G_CHUNKS // _GATHER_UNROLL, gather_body, 0)

    pltpu.sync_copy(out_v, out_hbm.at[pl.ds(off, _B_PER_W)])


def kernel(timesteps, W, base, w_ini, base_ini, min_diff):
    as1 = lambda x: jnp.asarray(x, jnp.float32).reshape(1)
    return _snr_lookup(timesteps, W, as1(w_ini), as1(min_diff),
                       as1(base_ini), as1(base[0]))


# speedup vs baseline: 4.7758x; 1.0172x over previous
import functools

import jax
import jax.numpy as jnp
from jax import lax
from jax.experimental import pallas as pl
from jax.experimental.pallas import tpu as pltpu
from jax.experimental.pallas import tpu_sc as plsc

_L = 16
_T = 1001
_T_PAD = 1024
_N_CHUNKS = _T_PAD // _L
_B = 16384
_NW = 32
_B_PER_W = _B // _NW
_G_CHUNKS = _B_PER_W // _L
_BUILD_UNROLL = 8
_GATHER_UNROLL = 8

_mesh = plsc.VectorSubcoreMesh(core_axis_name="c", subcore_axis_name="s")


@functools.partial(
    pl.kernel,
    mesh=_mesh,
    out_type=jax.ShapeDtypeStruct((_B,), jnp.float32),
    compiler_params=pltpu.CompilerParams(needs_layout_passes=False),
    scratch_types=[
        pltpu.VMEM((_T_PAD,), jnp.float32),
        pltpu.VMEM((2 * _L,), jnp.float32),
        pltpu.VMEM((_B_PER_W,), jnp.int32),
        pltpu.VMEM((_B_PER_W,), jnp.float32),
        pltpu.SemaphoreType.DMA,
        pltpu.SemaphoreType.DMA,
        pltpu.SemaphoreType.DMA,
    ],
)
def _snr_lookup(ts_hbm, w_hbm, wini_hbm, md_hbm, bini_hbm, base_hbm, out_hbm,
                tab_v, scal_v, idx_v, out_v, sem0, sem1, sem2):
    wid = lax.axis_index("s") * 2 + lax.axis_index("c")
    off = wid * _B_PER_W
    cw = pltpu.async_copy(w_hbm, tab_v.at[pl.ds(0, _T)], sem0)
    c0 = pltpu.async_copy(wini_hbm, scal_v.at[pl.ds(0, 1)], sem1)
    c1 = pltpu.async_copy(md_hbm, scal_v.at[pl.ds(8, 1)], sem1)
    c2 = pltpu.async_copy(bini_hbm, scal_v.at[pl.ds(16, 1)], sem1)
    c3 = pltpu.async_copy(base_hbm, scal_v.at[pl.ds(24, 1)], sem1)
    ci = pltpu.async_copy(ts_hbm.at[pl.ds(off, _B_PER_W)], idx_v, sem2)

    c0.wait()
    c1.wait()
    c2.wait()
    c3.wait()
    sv0 = scal_v[pl.ds(0, _L)]
    sv1 = scal_v[pl.ds(_L, _L)]
    wini = lax.broadcast(sv0[0], (_L,))
    md = lax.broadcast(sv0[8], (_L,))
    carry0 = lax.broadcast(-(sv1[0] + sv1[8]), (_L,))
    cw.wait()

    @plsc.parallel_loop(0, _N_CHUNKS, 1, unroll=_BUILD_UNROLL, carry=carry0)
    def _build(i, carry):
        x = tab_v[pl.ds(i * _L, _L)] + wini
        s = 1.0 / (1.0 + jnp.exp(-x)) + md
        tab_v[pl.ds(i * _L, _L)] = plsc.cumsum(s) + carry
        return carry + lax.broadcast(jnp.sum(s), (_L,))

    ci.wait()

    @plsc.parallel_loop(0, _G_CHUNKS, 1, unroll=_GATHER_UNROLL)
    def _gather(j):
        ids = idx_v[pl.ds(j * _L, _L)]
        out_v[pl.ds(j * _L, _L)] = plsc.load_gather(tab_v, [ids])

    pltpu.sync_copy(out_v, out_hbm.at[pl.ds(off, _B_PER_W)])


def kernel(timesteps, W, base, w_ini, base_ini, min_diff):
    as1 = lambda x: jnp.asarray(x, jnp.float32).reshape(1)
    return _snr_lookup(timesteps, W, as1(w_ini), as1(min_diff),
                       as1(base_ini), as1(base[0]))


# speedup vs baseline: 4.7991x; 1.0049x over previous
import functools

import jax
import jax.numpy as jnp
from jax import lax
from jax.experimental import pallas as pl
from jax.experimental.pallas import tpu as pltpu
from jax.experimental.pallas import tpu_sc as plsc

_L = 16
_T = 1001
_T_PAD = 1024
_N_CHUNKS = _T_PAD // _L
_B = 16384
_NW = 32
_B_PER_W = _B // _NW
_G_CHUNKS = _B_PER_W // _L
_BUILD_UNROLL = 4
_GATHER_UNROLL = 4

_mesh = plsc.VectorSubcoreMesh(core_axis_name="c", subcore_axis_name="s")


@functools.partial(
    pl.kernel,
    mesh=_mesh,
    out_type=jax.ShapeDtypeStruct((_B,), jnp.float32),
    compiler_params=pltpu.CompilerParams(needs_layout_passes=False),
    scratch_types=[
        pltpu.VMEM((_T_PAD,), jnp.float32),
        pltpu.VMEM((2 * _L,), jnp.float32),
        pltpu.VMEM((_B_PER_W,), jnp.int32),
        pltpu.VMEM((_B_PER_W,), jnp.float32),
        pltpu.SemaphoreType.DMA,
        pltpu.SemaphoreType.DMA,
        pltpu.SemaphoreType.DMA,
    ],
)
def _snr_lookup(ts_hbm, w_hbm, wini_hbm, md_hbm, bini_hbm, base_hbm, out_hbm,
                tab_v, scal_v, idx_v, out_v, sem0, sem1, sem2):
    wid = lax.axis_index("s") * 2 + lax.axis_index("c")
    off = wid * _B_PER_W
    cw = pltpu.async_copy(w_hbm, tab_v.at[pl.ds(0, _T)], sem0)
    c0 = pltpu.async_copy(wini_hbm, scal_v.at[pl.ds(0, 1)], sem1)
    c1 = pltpu.async_copy(md_hbm, scal_v.at[pl.ds(8, 1)], sem1)
    c2 = pltpu.async_copy(bini_hbm, scal_v.at[pl.ds(16, 1)], sem1)
    c3 = pltpu.async_copy(base_hbm, scal_v.at[pl.ds(24, 1)], sem1)
    ci = pltpu.async_copy(ts_hbm.at[pl.ds(off, _B_PER_W)], idx_v, sem2)

    c0.wait()
    c1.wait()
    c2.wait()
    c3.wait()
    sv0 = scal_v[pl.ds(0, _L)]
    sv1 = scal_v[pl.ds(_L, _L)]
    wini = lax.broadcast(sv0[0], (_L,))
    md = lax.broadcast(sv0[8], (_L,))
    carry0 = lax.broadcast(-(sv1[0] + sv1[8]), (_L,))
    cw.wait()

    @plsc.parallel_loop(0, _N_CHUNKS, 1, unroll=_BUILD_UNROLL, carry=carry0)
    def _build(i, carry):
        x = tab_v[pl.ds(i * _L, _L)] + wini
        s = 1.0 / (1.0 + jnp.exp(-x)) + md
        tab_v[pl.ds(i * _L, _L)] = plsc.cumsum(s) + carry
        return carry + lax.broadcast(jnp.sum(s), (_L,))

    ci.wait()

    @plsc.parallel_loop(0, _G_CHUNKS, 1, unroll=_GATHER_UNROLL)
    def _gather(j):
        ids = idx_v[pl.ds(j * _L, _L)]
        out_v[pl.ds(j * _L, _L)] = plsc.load_gather(tab_v, [ids])

    pltpu.sync_copy(out_v, out_hbm.at[pl.ds(off, _B_PER_W)])


def kernel(timesteps, W, base, w_ini, base_ini, min_diff):
    as1 = lambda x: jnp.asarray(x, jnp.float32).reshape(1)
    return _snr_lookup(timesteps, W, as1(w_ini), as1(min_diff),
                       as1(base_ini), as1(base[0]))
